# baseline (speedup 1.0000x reference)
import functools

import jax
import jax.numpy as jnp
from jax import lax
from jax.experimental import pallas as pl
from jax.experimental.pallas import tpu as pltpu
from jax.experimental.pallas import tpu_sc as plsc

EDGES = 320000
RPAD = 512
BPAD = 10240
D = 128
KBLK = 1024
NK = BPAD // KBLK

NC = 2
NS = 16
NW = NC * NS

CHUNK = 8000
NCH = EDGES // CHUNK
NPAIR = NCH // 2
PER_TILE = EDGES // NS
CH0 = 4000
NCH0 = PER_TILE // CH0
SLAB_R = 8
SLAB_W = SLAB_R * BPAD
SLAB_PAD = SLAB_W + 16
NSLAB_PER_TILE = (RPAD // SLAB_R) // NS

GB = BPAD // NW

_MESH = dict(core_axis_name="c", subcore_axis_name="s", num_cores=NC,
             num_subcores=NS)


@functools.partial(
    pl.kernel,
    out_type=jax.ShapeDtypeStruct((BPAD, D), jnp.float32),
    mesh=plsc.VectorSubcoreMesh(**_MESH),
    scratch_types=[
        pltpu.VMEM((GB,), jnp.int32),
        pltpu.VMEM((GB, D), jnp.float32),
        pltpu.SemaphoreType.DMA,
    ],
    compiler_params=pltpu.CompilerParams(needs_layout_passes=False),
)
def _sc_gather(table, idx, out, idx_v, rows_v, sem):
    wid = lax.axis_index("s") * NC + lax.axis_index("c")
    base = wid * GB
    pltpu.sync_copy(idx.at[pl.ds(base, GB)], idx_v)
    pltpu.async_copy(table.at[idx_v], rows_v, sem).wait()
    pltpu.sync_copy(rows_v, out.at[pl.ds(base, GB)])


@functools.partial(
    pl.kernel,
    out_type=[
        jax.ShapeDtypeStruct((NC * EDGES,), jnp.int32),
        jax.ShapeDtypeStruct((2 * RPAD * BPAD,), jnp.float32),
    ],
    mesh=plsc.VectorSubcoreMesh(**_MESH),
    scratch_types=[
        pltpu.VMEM((CHUNK,), jnp.int32),
        pltpu.VMEM((CHUNK,), jnp.int32),
        pltpu.VMEM((SLAB_PAD,), jnp.float32),
        pltpu.SemaphoreType.DMA,
        pltpu.SemaphoreType.DMA,
    ],
    compiler_params=pltpu.CompilerParams(needs_layout_passes=False),
)
def _sc_count(etype, eidx, keys, cflat, kb0, kb1, slab, sem0, sem1):
    cid = lax.axis_index("c")
    sid = lax.axis_index("s")
    ones = jnp.ones((16,), jnp.float32)
    limit_u = plsc.bitcast(SLAB_W + lax.iota(jnp.int32, 16), jnp.uint32)
    kbase = cid * EDGES

    for c0 in range(NCH0):
        base = sid * PER_TILE + c0 * CH0
        pltpu.sync_copy(etype.at[pl.ds(base, CH0)], kb0.at[pl.ds(0, CH0)])
        pltpu.sync_copy(eidx.at[pl.ds(kbase + base, CH0)],
                        kb1.at[pl.ds(0, CH0)])

        def p0(i, carry):
            s16 = pl.ds(i * 16, 16)
            kb1[s16] = kb0[s16] * BPAD + kb1[s16]
            return carry

        lax.fori_loop(0, CH0 // 16, p0, 0, unroll=10)
        pltpu.sync_copy(kb1.at[pl.ds(0, CH0)],
                        keys.at[pl.ds(kbase + base, CH0)])
    plsc.subcore_barrier()

    for j in range(NSLAB_PER_TILE):
        rb = sid * NSLAB_PER_TILE + j
        lo_key = rb * SLAB_W

        def zero(i, carry):
            slab[pl.ds(i * 16, 16)] = jnp.zeros((16,), jnp.float32)
            return carry

        lax.fori_loop(0, SLAB_W // 16, zero, 0, unroll=8)

        def process(kb):
            def inner(i, c2):
                kv = kb[pl.ds(i * 16, 16)]
                tu = plsc.bitcast(kv - lo_key, jnp.uint32)
                m = tu < jnp.uint32(SLAB_W)
                idx = plsc.bitcast(jnp.minimum(tu, limit_u), jnp.int32)
                plsc.addupdate_scatter(slab, [idx], ones, mask=m)
                return c2

            lax.fori_loop(0, CHUNK // 16, inner, 0, unroll=10)

        pltpu.async_copy(keys.at[pl.ds(kbase, CHUNK)], kb0, sem0)

        def pair(p, carry):
            off1 = kbase + (2 * p + 1) * CHUNK
            pltpu.async_copy(keys.at[pl.ds(off1, CHUNK)], kb1, sem1)
            pltpu.make_async_copy(keys.at[pl.ds(kbase, CHUNK)], kb0,
                                  sem0).wait()
            process(kb0)

            @pl.when(p < NPAIR - 1)
            def _refill():
                off2 = kbase + (2 * p + 2) * CHUNK
                pltpu.async_copy(keys.at[pl.ds(off2, CHUNK)], kb0, sem0)

            pltpu.make_async_copy(keys.at[pl.ds(kbase, CHUNK)], kb1,
                                  sem1).wait()
            process(kb1)
            return carry

        lax.fori_loop(0, NPAIR, pair, 0)
        row0 = (cid * RPAD + rb * SLAB_R) * BPAD
        pltpu.sync_copy(slab.at[pl.ds(0, SLAB_W)],
                        cflat.at[pl.ds(row0, SLAB_W)])


def _tc_body(c_ref, g_ref, h0_ref, wih_ref, whh_ref, bih_ref, bhh_ref,
             out_ref, sums_ref, cnt_ref):
    k = pl.program_id(1)

    @pl.when(k == 0)
    def _init():
        sums_ref[...] = jnp.zeros_like(sums_ref)
        cnt_ref[...] = jnp.zeros_like(cnt_ref)

    cf = c_ref[0].astype(jnp.float32)
    sums_ref[...] += lax.dot_general(
        cf, g_ref[...], (((1,), (0,)), ((), ())),
        preferred_element_type=jnp.float32)
    cnt_ref[...] += jnp.broadcast_to(
        jnp.sum(cf, axis=1, keepdims=True), (RPAD, D))

    @pl.when(k == NK - 1)
    def _finish():
        mean = sums_ref[...] / jnp.maximum(cnt_ref[...], 1.0)
        h = h0_ref[0]
        gi = lax.dot_general(mean, wih_ref[...], (((1,), (1,)), ((), ())),
                             preferred_element_type=jnp.float32) + bih_ref[...]
        gh = lax.dot_general(h, whh_ref[...], (((1,), (1,)), ((), ())),
                             preferred_element_type=jnp.float32) + bhh_ref[...]
        r = jax.nn.sigmoid(gi[:, :D] + gh[:, :D])
        z = jax.nn.sigmoid(gi[:, D:2 * D] + gh[:, D:2 * D])
        n = jnp.tanh(gi[:, 2 * D:] + r * gh[:, 2 * D:])
        out_ref[0] = (1.0 - z) * n + z * h


def _tc_call(c, g, h0, w_ih, w_hh, b_ih, b_hh):
    return pl.pallas_call(
        _tc_body,
        grid=(2, NK),
        in_specs=[
            pl.BlockSpec((1, RPAD, KBLK), lambda s, k: (s, 0, k)),
            pl.BlockSpec((KBLK, D), lambda s, k: (k, 0)),
            pl.BlockSpec((1, RPAD, D), lambda s, k: (s, 0, 0)),
            pl.BlockSpec((3 * D, D), lambda s, k: (0, 0)),
            pl.BlockSpec((3 * D, D), lambda s, k: (0, 0)),
            pl.BlockSpec((1, 3 * D), lambda s, k: (0, 0)),
            pl.BlockSpec((1, 3 * D), lambda s, k: (0, 0)),
        ],
        out_specs=pl.BlockSpec((1, RPAD, D), lambda s, k: (s, 0, 0)),
        out_shape=jax.ShapeDtypeStruct((2, RPAD, D), jnp.float32),
        scratch_shapes=[
            pltpu.VMEM((RPAD, D), jnp.float32),
            pltpu.VMEM((RPAD, D), jnp.float32),
        ],
        compiler_params=pltpu.CompilerParams(
            dimension_semantics=("arbitrary", "arbitrary")),
    )(c, g, h0, w_ih, w_hh, b_ih, b_hh)


def kernel(edge_index, edge_type, node_id, dynamic_relation_emb,
           static_entity_emb, W_ih, W_hh, b_ih, b_hh):
    edge_type = edge_type.astype(jnp.int32)
    eidx = edge_index.astype(jnp.int32).reshape(-1)
    node_id = node_id.astype(jnp.int32)

    nid_pad = jnp.pad(node_id, (0, BPAD - node_id.shape[0]))
    g = _sc_gather(static_entity_emb, nid_pad)
    _, cflat = _sc_count(edge_type, eidx)
    c = cflat.reshape(2, RPAD, BPAD)

    h_src = dynamic_relation_emb[:, 0, :, 0]
    h_dst = dynamic_relation_emb[:, 0, :, 1]
    nrel = h_src.shape[0]
    h0 = jnp.stack([
        jnp.pad(h_src, ((0, RPAD - nrel), (0, 0))),
        jnp.pad(h_dst, ((0, RPAD - nrel), (0, 0))),
    ])

    out = _tc_call(c, g, h0, W_ih, W_hh,
                   b_ih.reshape(1, 3 * D), b_hh.reshape(1, 3 * D))
    h_src_new = out[0, :nrel]
    h_dst_new = out[1, :nrel]
    return jnp.concatenate(
        [h_src_new[:, None, :, None], h_dst_new[:, None, :, None]], axis=-1)


# speedup vs baseline: 2.5675x; 2.5675x over previous
import functools

import jax
import jax.numpy as jnp
from jax import lax
from jax.experimental import pallas as pl
from jax.experimental.pallas import tpu as pltpu
from jax.experimental.pallas import tpu_sc as plsc

EDGES = 320000
RPAD = 512
BPAD = 10240
D = 128
KBLK = 1024
NK = BPAD // KBLK

NC = 2
NS = 16
NW = NC * NS

CHUNK = 8000
NCH = EDGES // CHUNK
NPAIR = NCH // 2
PER_TILE = EDGES // NS
CH0 = 4000
NCH0 = PER_TILE // CH0
SLAB_R = 8
SLAB_W = SLAB_R * BPAD
SLAB_PAD = SLAB_W + 16
NSLAB_PER_TILE = (RPAD // SLAB_R) // NS

GB = BPAD // NW

_MESH = dict(core_axis_name="c", subcore_axis_name="s", num_cores=NC,
             num_subcores=NS)


@functools.partial(
    pl.kernel,
    out_type=jax.ShapeDtypeStruct((BPAD, D), jnp.float32),
    mesh=plsc.VectorSubcoreMesh(**_MESH),
    scratch_types=[
        pltpu.VMEM((GB,), jnp.int32),
        pltpu.VMEM((GB, D), jnp.float32),
        pltpu.SemaphoreType.DMA,
    ],
    compiler_params=pltpu.CompilerParams(needs_layout_passes=False),
)
def _sc_gather(table, idx, out, idx_v, rows_v, sem):
    wid = lax.axis_index("s") * NC + lax.axis_index("c")
    base = wid * GB
    pltpu.sync_copy(idx.at[pl.ds(base, GB)], idx_v)
    pltpu.async_copy(table.at[idx_v], rows_v, sem).wait()
    pltpu.sync_copy(rows_v, out.at[pl.ds(base, GB)])


@functools.partial(
    pl.kernel,
    out_type=[
        jax.ShapeDtypeStruct((NC * EDGES,), jnp.int32),
        jax.ShapeDtypeStruct((2 * RPAD * BPAD,), jnp.float32),
    ],
    mesh=plsc.VectorSubcoreMesh(**_MESH),
    scratch_types=[
        pltpu.VMEM((CHUNK,), jnp.int32),
        pltpu.VMEM((CHUNK,), jnp.int32),
        pltpu.VMEM((SLAB_PAD,), jnp.float32),
        pltpu.SemaphoreType.DMA,
        pltpu.SemaphoreType.DMA,
    ],
    compiler_params=pltpu.CompilerParams(needs_layout_passes=False),
)
def _sc_count(etype, eidx, keys, cflat, kb0, kb1, slab, sem0, sem1):
    cid = lax.axis_index("c")
    sid = lax.axis_index("s")
    ones = jnp.ones((16,), jnp.float32)
    limit_u = plsc.bitcast(SLAB_W + lax.iota(jnp.int32, 16), jnp.uint32)
    kbase = cid * EDGES

    for c0 in range(NCH0):
        base = sid * PER_TILE + c0 * CH0
        pltpu.sync_copy(etype.at[pl.ds(base, CH0)], kb0.at[pl.ds(0, CH0)])
        pltpu.sync_copy(eidx.at[pl.ds(kbase + base, CH0)],
                        kb1.at[pl.ds(0, CH0)])

        @plsc.parallel_loop(0, CH0, 16, unroll=8)
        def _p0(i):
            s16 = pl.ds(i, 16)
            kb1[s16] = kb0[s16] * BPAD + kb1[s16]
        pltpu.sync_copy(kb1.at[pl.ds(0, CH0)],
                        keys.at[pl.ds(kbase + base, CH0)])
    plsc.subcore_barrier()

    for j in range(NSLAB_PER_TILE):
        rb = sid * NSLAB_PER_TILE + j
        lo_key = rb * SLAB_W

        @plsc.parallel_loop(0, SLAB_W, 16, unroll=8)
        def _zero(i):
            slab[pl.ds(i, 16)] = jnp.zeros((16,), jnp.float32)

        def process(kb):
            @plsc.parallel_loop(0, CHUNK, 16, unroll=8)
            def _inner(i):
                kv = kb[pl.ds(i, 16)]
                tu = plsc.bitcast(kv - lo_key, jnp.uint32)
                idx = plsc.bitcast(jnp.minimum(tu, limit_u), jnp.int32)
                plsc.addupdate_scatter(slab, [idx], ones)

        pltpu.async_copy(keys.at[pl.ds(kbase, CHUNK)], kb0, sem0)

        def pair(p, carry):
            off1 = kbase + (2 * p + 1) * CHUNK
            pltpu.async_copy(keys.at[pl.ds(off1, CHUNK)], kb1, sem1)
            pltpu.make_async_copy(keys.at[pl.ds(kbase, CHUNK)], kb0,
                                  sem0).wait()
            process(kb0)

            @pl.when(p < NPAIR - 1)
            def _refill():
                off2 = kbase + (2 * p + 2) * CHUNK
                pltpu.async_copy(keys.at[pl.ds(off2, CHUNK)], kb0, sem0)

            pltpu.make_async_copy(keys.at[pl.ds(kbase, CHUNK)], kb1,
                                  sem1).wait()
            process(kb1)
            return carry

        lax.fori_loop(0, NPAIR, pair, 0)
        row0 = (cid * RPAD + rb * SLAB_R) * BPAD
        pltpu.sync_copy(slab.at[pl.ds(0, SLAB_W)],
                        cflat.at[pl.ds(row0, SLAB_W)])


def _tc_body(c_ref, g_ref, h0_ref, wih_ref, whh_ref, bih_ref, bhh_ref,
             out_ref, sums_ref, cnt_ref):
    k = pl.program_id(1)

    @pl.when(k == 0)
    def _init():
        sums_ref[...] = jnp.zeros_like(sums_ref)
        cnt_ref[...] = jnp.zeros_like(cnt_ref)

    cf = c_ref[0].astype(jnp.float32)
    sums_ref[...] += lax.dot_general(
        cf, g_ref[...], (((1,), (0,)), ((), ())),
        preferred_element_type=jnp.float32)
    cnt_ref[...] += jnp.broadcast_to(
        jnp.sum(cf, axis=1, keepdims=True), (RPAD, D))

    @pl.when(k == NK - 1)
    def _finish():
        mean = sums_ref[...] / jnp.maximum(cnt_ref[...], 1.0)
        h = h0_ref[0]
        gi = lax.dot_general(mean, wih_ref[...], (((1,), (1,)), ((), ())),
                             preferred_element_type=jnp.float32) + bih_ref[...]
        gh = lax.dot_general(h, whh_ref[...], (((1,), (1,)), ((), ())),
                             preferred_element_type=jnp.float32) + bhh_ref[...]
        r = jax.nn.sigmoid(gi[:, :D] + gh[:, :D])
        z = jax.nn.sigmoid(gi[:, D:2 * D] + gh[:, D:2 * D])
        n = jnp.tanh(gi[:, 2 * D:] + r * gh[:, 2 * D:])
        out_ref[0] = (1.0 - z) * n + z * h


def _tc_call(c, g, h0, w_ih, w_hh, b_ih, b_hh):
    return pl.pallas_call(
        _tc_body,
        grid=(2, NK),
        in_specs=[
            pl.BlockSpec((1, RPAD, KBLK), lambda s, k: (s, 0, k)),
            pl.BlockSpec((KBLK, D), lambda s, k: (k, 0)),
            pl.BlockSpec((1, RPAD, D), lambda s, k: (s, 0, 0)),
            pl.BlockSpec((3 * D, D), lambda s, k: (0, 0)),
            pl.BlockSpec((3 * D, D), lambda s, k: (0, 0)),
            pl.BlockSpec((1, 3 * D), lambda s, k: (0, 0)),
            pl.BlockSpec((1, 3 * D), lambda s, k: (0, 0)),
        ],
        out_specs=pl.BlockSpec((1, RPAD, D), lambda s, k: (s, 0, 0)),
        out_shape=jax.ShapeDtypeStruct((2, RPAD, D), jnp.float32),
        scratch_shapes=[
            pltpu.VMEM((RPAD, D), jnp.float32),
            pltpu.VMEM((RPAD, D), jnp.float32),
        ],
        compiler_params=pltpu.CompilerParams(
            dimension_semantics=("arbitrary", "arbitrary")),
    )(c, g, h0, w_ih, w_hh, b_ih, b_hh)


def kernel(edge_index, edge_type, node_id, dynamic_relation_emb,
           static_entity_emb, W_ih, W_hh, b_ih, b_hh):
    edge_type = edge_type.astype(jnp.int32)
    eidx = edge_index.astype(jnp.int32).reshape(-1)
    node_id = node_id.astype(jnp.int32)

    nid_pad = jnp.pad(node_id, (0, BPAD - node_id.shape[0]))
    g = _sc_gather(static_entity_emb, nid_pad)
    _, cflat = _sc_count(edge_type, eidx)
    c = cflat.reshape(2, RPAD, BPAD)

    h_src = dynamic_relation_emb[:, 0, :, 0]
    h_dst = dynamic_relation_emb[:, 0, :, 1]
    nrel = h_src.shape[0]
    h0 = jnp.stack([
        jnp.pad(h_src, ((0, RPAD - nrel), (0, 0))),
        jnp.pad(h_dst, ((0, RPAD - nrel), (0, 0))),
    ])

    out = _tc_call(c, g, h0, W_ih, W_hh,
                   b_ih.reshape(1, 3 * D), b_hh.reshape(1, 3 * D))
    h_src_new = out[0, :nrel]
    h_dst_new = out[1, :nrel]
    return jnp.concatenate(
        [h_src_new[:, None, :, None], h_dst_new[:, None, :, None]], axis=-1)


# speedup vs baseline: 2.5991x; 1.0123x over previous
import functools

import jax
import jax.numpy as jnp
from jax import lax
from jax.experimental import pallas as pl
from jax.experimental.pallas import tpu as pltpu
from jax.experimental.pallas import tpu_sc as plsc

EDGES = 320000
RPAD = 512
BPAD = 10240
D = 128
KBLK = 1024
NK = BPAD // KBLK

NC = 2
NS = 16
NW = NC * NS

CHUNK = 8000
NCH = EDGES // CHUNK
NPAIR = NCH // 2
PER_TILE = EDGES // NS
CH0 = 4000
NCH0 = PER_TILE // CH0
SLAB_R = 8
SLAB_W = SLAB_R * BPAD
SLAB_PAD = SLAB_W + 16
NSLAB_PER_TILE = (RPAD // SLAB_R) // NS

GB = BPAD // NW
GH = GB // 2

_MESH = dict(core_axis_name="c", subcore_axis_name="s", num_cores=NC,
             num_subcores=NS)


@functools.partial(
    pl.kernel,
    out_type=[
        jax.ShapeDtypeStruct((NC * EDGES,), jnp.int32),
        jax.ShapeDtypeStruct((2 * RPAD * BPAD,), jnp.float32),
        jax.ShapeDtypeStruct((BPAD, D), jnp.float32),
    ],
    mesh=plsc.VectorSubcoreMesh(**_MESH),
    scratch_types=[
        pltpu.VMEM((CHUNK,), jnp.int32),
        pltpu.VMEM((CHUNK,), jnp.int32),
        pltpu.VMEM((SLAB_PAD,), jnp.float32),
        pltpu.VMEM((GB,), jnp.int32),
        pltpu.VMEM((GH, D), jnp.float32),
        pltpu.SemaphoreType.DMA,
        pltpu.SemaphoreType.DMA,
        pltpu.SemaphoreType.DMA,
    ],
    compiler_params=pltpu.CompilerParams(needs_layout_passes=False),
)
def _sc_count(etype, eidx, table, nid, keys, cflat, g,
              kb0, kb1, slab, idx_v, rows_v, sem0, sem1, semg):
    cid = lax.axis_index("c")
    sid = lax.axis_index("s")
    wid = sid * NC + cid
    gbase = wid * GB
    pltpu.sync_copy(nid.at[pl.ds(gbase, GB)], idx_v)
    pltpu.async_copy(table.at[idx_v.at[pl.ds(0, GH)]], rows_v, semg)
    ones = jnp.ones((16,), jnp.float32)
    limit_u = plsc.bitcast(SLAB_W + lax.iota(jnp.int32, 16), jnp.uint32)
    kbase = cid * EDGES

    for c0 in range(NCH0):
        base = sid * PER_TILE + c0 * CH0
        pltpu.sync_copy(etype.at[pl.ds(base, CH0)], kb0.at[pl.ds(0, CH0)])
        pltpu.sync_copy(eidx.at[pl.ds(kbase + base, CH0)],
                        kb1.at[pl.ds(0, CH0)])

        @plsc.parallel_loop(0, CH0, 16, unroll=8)
        def _p0(i):
            s16 = pl.ds(i, 16)
            kb1[s16] = kb0[s16] * BPAD + kb1[s16]
        pltpu.sync_copy(kb1.at[pl.ds(0, CH0)],
                        keys.at[pl.ds(kbase + base, CH0)])
    pltpu.make_async_copy(table.at[idx_v.at[pl.ds(0, GH)]], rows_v, semg).wait()
    pltpu.sync_copy(rows_v, g.at[pl.ds(gbase, GH)])
    pltpu.async_copy(table.at[idx_v.at[pl.ds(GH, GH)]], rows_v, semg)
    plsc.subcore_barrier()

    for j in range(NSLAB_PER_TILE):
        rb = sid * NSLAB_PER_TILE + j
        lo_key = rb * SLAB_W

        @plsc.parallel_loop(0, SLAB_W, 16, unroll=8)
        def _zero(i):
            slab[pl.ds(i, 16)] = jnp.zeros((16,), jnp.float32)

        def process(kb):
            @plsc.parallel_loop(0, CHUNK, 16, unroll=8)
            def _inner(i):
                kv = kb[pl.ds(i, 16)]
                tu = plsc.bitcast(kv - lo_key, jnp.uint32)
                idx = plsc.bitcast(jnp.minimum(tu, limit_u), jnp.int32)
                plsc.addupdate_scatter(slab, [idx], ones)

        pltpu.async_copy(keys.at[pl.ds(kbase, CHUNK)], kb0, sem0)

        def pair(p, carry):
            off1 = kbase + (2 * p + 1) * CHUNK
            pltpu.async_copy(keys.at[pl.ds(off1, CHUNK)], kb1, sem1)
            pltpu.make_async_copy(keys.at[pl.ds(kbase, CHUNK)], kb0,
                                  sem0).wait()
            process(kb0)

            @pl.when(p < NPAIR - 1)
            def _refill():
                off2 = kbase + (2 * p + 2) * CHUNK
                pltpu.async_copy(keys.at[pl.ds(off2, CHUNK)], kb0, sem0)

            pltpu.make_async_copy(keys.at[pl.ds(kbase, CHUNK)], kb1,
                                  sem1).wait()
            process(kb1)
            return carry

        lax.fori_loop(0, NPAIR, pair, 0)
        row0 = (cid * RPAD + rb * SLAB_R) * BPAD
        pltpu.sync_copy(slab.at[pl.ds(0, SLAB_W)],
                        cflat.at[pl.ds(row0, SLAB_W)])
        if j == 0:
            pltpu.make_async_copy(table.at[idx_v.at[pl.ds(GH, GH)]], rows_v,
                                  semg).wait()
            pltpu.sync_copy(rows_v, g.at[pl.ds(gbase + GH, GH)])


def _tc_body(c_ref, g_ref, h0_ref, wih_ref, whh_ref, bih_ref, bhh_ref,
             out_ref, sums_ref, cnt_ref):
    k = pl.program_id(1)

    @pl.when(k == 0)
    def _init():
        sums_ref[...] = jnp.zeros_like(sums_ref)
        cnt_ref[...] = jnp.zeros_like(cnt_ref)

    cf = c_ref[0].astype(jnp.float32)
    sums_ref[...] += lax.dot_general(
        cf, g_ref[...], (((1,), (0,)), ((), ())),
        preferred_element_type=jnp.float32)
    cnt_ref[...] += jnp.broadcast_to(
        jnp.sum(cf, axis=1, keepdims=True), (RPAD, D))

    @pl.when(k == NK - 1)
    def _finish():
        mean = sums_ref[...] / jnp.maximum(cnt_ref[...], 1.0)
        h = h0_ref[0]
        gi = lax.dot_general(mean, wih_ref[...], (((1,), (1,)), ((), ())),
                             preferred_element_type=jnp.float32) + bih_ref[...]
        gh = lax.dot_general(h, whh_ref[...], (((1,), (1,)), ((), ())),
                             preferred_element_type=jnp.float32) + bhh_ref[...]
        r = jax.nn.sigmoid(gi[:, :D] + gh[:, :D])
        z = jax.nn.sigmoid(gi[:, D:2 * D] + gh[:, D:2 * D])
        n = jnp.tanh(gi[:, 2 * D:] + r * gh[:, 2 * D:])
        out_ref[0] = (1.0 - z) * n + z * h


def _tc_call(c, g, h0, w_ih, w_hh, b_ih, b_hh):
    return pl.pallas_call(
        _tc_body,
        grid=(2, NK),
        in_specs=[
            pl.BlockSpec((1, RPAD, KBLK), lambda s, k: (s, 0, k)),
            pl.BlockSpec((KBLK, D), lambda s, k: (k, 0)),
            pl.BlockSpec((1, RPAD, D), lambda s, k: (s, 0, 0)),
            pl.BlockSpec((3 * D, D), lambda s, k: (0, 0)),
            pl.BlockSpec((3 * D, D), lambda s, k: (0, 0)),
            pl.BlockSpec((1, 3 * D), lambda s, k: (0, 0)),
            pl.BlockSpec((1, 3 * D), lambda s, k: (0, 0)),
        ],
        out_specs=pl.BlockSpec((1, RPAD, D), lambda s, k: (s, 0, 0)),
        out_shape=jax.ShapeDtypeStruct((2, RPAD, D), jnp.float32),
        scratch_shapes=[
            pltpu.VMEM((RPAD, D), jnp.float32),
            pltpu.VMEM((RPAD, D), jnp.float32),
        ],
        compiler_params=pltpu.CompilerParams(
            dimension_semantics=("arbitrary", "arbitrary")),
    )(c, g, h0, w_ih, w_hh, b_ih, b_hh)


def kernel(edge_index, edge_type, node_id, dynamic_relation_emb,
           static_entity_emb, W_ih, W_hh, b_ih, b_hh):
    edge_type = edge_type.astype(jnp.int32)
    eidx = edge_index.astype(jnp.int32).reshape(-1)
    node_id = node_id.astype(jnp.int32)

    nid_pad = jnp.pad(node_id, (0, BPAD - node_id.shape[0]))
    _, cflat, g = _sc_count(edge_type, eidx, static_entity_emb, nid_pad)
    c = cflat.reshape(2, RPAD, BPAD)

    h_src = dynamic_relation_emb[:, 0, :, 0]
    h_dst = dynamic_relation_emb[:, 0, :, 1]
    nrel = h_src.shape[0]
    h0 = jnp.stack([
        jnp.pad(h_src, ((0, RPAD - nrel), (0, 0))),
        jnp.pad(h_dst, ((0, RPAD - nrel), (0, 0))),
    ])

    out = _tc_call(c, g, h0, W_ih, W_hh,
                   b_ih.reshape(1, 3 * D), b_hh.reshape(1, 3 * D))
    h_src_new = out[0, :nrel]
    h_dst_new = out[1, :nrel]
    return jnp.concatenate(
        [h_src_new[:, None, :, None], h_dst_new[:, None, :, None]], axis=-1)
